# baseline (speedup 1.0000x reference)
import functools

import jax
import jax.numpy as jnp
from jax import lax
from jax.experimental import pallas as pl
from jax.experimental.pallas import tpu as pltpu
from jax.experimental.pallas import tpu_sc as plsc

BSZ = 64
N_DIM = 128
N_DATA = 100000
K = 8192
T = 0.07
M = 0.5

BLK = 16384
NBLK = (N_DATA + BLK - 1) // BLK
KP = 8208
CHUNKS = KP // 16
NC = 2
NS = 16
B_PER_W = BSZ // (NC * NS)


def _scores_body(x_ref, m_ref, s_ref, r_ref):
    dn = (((1,), (1,)), ((), ()))
    s_ref[...] = lax.dot_general(x_ref[...], m_ref[...], dn,
                                 preferred_element_type=jnp.float32)
    r_ref[...] = m_ref[...][:, None, :]


_scores_call = pl.pallas_call(
    _scores_body,
    grid=(NBLK,),
    in_specs=[
        pl.BlockSpec((BSZ, N_DIM), lambda i: (0, 0)),
        pl.BlockSpec((BLK, N_DIM), lambda i: (i, 0)),
    ],
    out_specs=[
        pl.BlockSpec((BSZ, BLK), lambda i: (0, i)),
        pl.BlockSpec((BLK, 1, N_DIM), lambda i: (i, 0, 0)),
    ],
    out_shape=(
        jax.ShapeDtypeStruct((BSZ, N_DATA), jnp.float32),
        jax.ShapeDtypeStruct((N_DATA, 1, N_DIM), jnp.float32),
    ),
)


_sc_mesh = plsc.VectorSubcoreMesh(
    core_axis_name="c", subcore_axis_name="s", num_cores=NC, num_subcores=NS)


@functools.partial(
    pl.kernel,
    out_type=jax.ShapeDtypeStruct((BSZ, KP), jnp.float32),
    mesh=_sc_mesh,
    compiler_params=pltpu.CompilerParams(needs_layout_passes=False),
    scratch_types=[
        pltpu.VMEM((N_DATA,), jnp.float32),
        pltpu.VMEM((KP,), jnp.int32),
        pltpu.VMEM((KP,), jnp.float32),
    ],
)
def _gather_kernel(s_hbm, idx_hbm, l_hbm, table_v, idx_v, out_v):
    wid = lax.axis_index("s") * NC + lax.axis_index("c")
    for r in range(B_PER_W):
        b = wid * B_PER_W + r
        pltpu.sync_copy(idx_hbm.at[b], idx_v)
        pltpu.sync_copy(s_hbm.at[b], table_v)

        def body(c, _):
            iv = idx_v[pl.ds(c * 16, 16)]
            out_v[pl.ds(c * 16, 16)] = plsc.load_gather(table_v, [iv]) / T
            return 0

        lax.fori_loop(0, CHUNKS, body, 0, unroll=8)
        pltpu.sync_copy(out_v, l_hbm.at[b])


def _update_body(y_ref, x1_ref, x2_ref, m1_ref, m2_ref, r1_ref, r2_ref,
                 o1_ref, o2_ref):
    del y_ref, r1_ref, r2_ref
    for x_ref, m_ref, o_ref in ((x1_ref, m1_ref, o1_ref),
                                (x2_ref, m2_ref, o2_ref)):
        w = m_ref[...] * M + x_ref[...] * (1.0 - M)
        n = jnp.sqrt(jnp.sum(w * w, axis=2, keepdims=True))
        o_ref[...] = w / jnp.clip(n, 1e-12, None)


_update_call = pl.pallas_call(
    _update_body,
    grid_spec=pltpu.PrefetchScalarGridSpec(
        num_scalar_prefetch=1,
        grid=(BSZ,),
        in_specs=[
            pl.BlockSpec((1, 1, N_DIM), lambda i, y: (i, 0, 0)),
            pl.BlockSpec((1, 1, N_DIM), lambda i, y: (i, 0, 0)),
            pl.BlockSpec((1, 1, N_DIM), lambda i, y: (y[i], 0, 0)),
            pl.BlockSpec((1, 1, N_DIM), lambda i, y: (y[i], 0, 0)),
            pl.BlockSpec((1, 1, N_DIM), lambda i, y: (y[i], 0, 0)),
            pl.BlockSpec((1, 1, N_DIM), lambda i, y: (y[i], 0, 0)),
        ],
        out_specs=[
            pl.BlockSpec((1, 1, N_DIM), lambda i, y: (y[i], 0, 0)),
            pl.BlockSpec((1, 1, N_DIM), lambda i, y: (y[i], 0, 0)),
        ],
    ),
    out_shape=(
        jax.ShapeDtypeStruct((N_DATA, 1, N_DIM), jnp.float32),
        jax.ShapeDtypeStruct((N_DATA, 1, N_DIM), jnp.float32),
    ),
    input_output_aliases={5: 0, 6: 1},
)


def kernel(x1, x2, y, memory_1, memory_2, idx):
    idx_pad = jnp.pad(idx.at[:, 0].set(y), ((0, 0), (0, KP - (K + 1))))
    scores1, raw2 = _scores_call(x1, memory_2)
    l1p = _gather_kernel(scores1, idx_pad)
    scores2, raw1 = _scores_call(x2, memory_1)
    l2p = _gather_kernel(scores2, idx_pad)
    new1, new2 = _update_call(
        y,
        x1.reshape(BSZ, 1, N_DIM),
        x2.reshape(BSZ, 1, N_DIM),
        memory_1.reshape(N_DATA, 1, N_DIM),
        memory_2.reshape(N_DATA, 1, N_DIM),
        raw1, raw2)
    labels = jnp.zeros((BSZ,), jnp.int32)
    return (l1p[:, :K + 1], l2p[:, :K + 1], labels,
            new1.reshape(N_DATA, N_DIM), new2.reshape(N_DATA, N_DIM))


# speedup vs baseline: 1.0045x; 1.0045x over previous
import functools

import jax
import jax.numpy as jnp
from jax import lax
from jax.experimental import pallas as pl
from jax.experimental.pallas import tpu as pltpu
from jax.experimental.pallas import tpu_sc as plsc

BSZ = 64
N_DIM = 128
N_DATA = 100000
K = 8192
T = 0.07
M = 0.5

BLK = 8192
NBLK = (N_DATA + BLK - 1) // BLK
KP = 8208
CHUNKS = KP // 16
NC = 2
NS = 16
B_PER_W = BSZ // (NC * NS)


def _scores_body(x_ref, m_ref, s_ref, r_ref):
    dn = (((1,), (1,)), ((), ()))
    s_ref[...] = lax.dot_general(x_ref[...], m_ref[...], dn,
                                 preferred_element_type=jnp.float32)
    r_ref[...] = m_ref[...][:, None, :]


_scores_call = pl.pallas_call(
    _scores_body,
    grid=(NBLK,),
    in_specs=[
        pl.BlockSpec((BSZ, N_DIM), lambda i: (0, 0)),
        pl.BlockSpec((BLK, N_DIM), lambda i: (i, 0)),
    ],
    out_specs=[
        pl.BlockSpec((BSZ, BLK), lambda i: (0, i)),
        pl.BlockSpec((BLK, 1, N_DIM), lambda i: (i, 0, 0)),
    ],
    out_shape=(
        jax.ShapeDtypeStruct((BSZ, N_DATA), jnp.float32),
        jax.ShapeDtypeStruct((N_DATA, 1, N_DIM), jnp.float32),
    ),
)


_sc_mesh = plsc.VectorSubcoreMesh(
    core_axis_name="c", subcore_axis_name="s", num_cores=NC, num_subcores=NS)


@functools.partial(
    pl.kernel,
    out_type=jax.ShapeDtypeStruct((BSZ, KP), jnp.float32),
    mesh=_sc_mesh,
    compiler_params=pltpu.CompilerParams(needs_layout_passes=False),
    scratch_types=[
        pltpu.VMEM((N_DATA,), jnp.float32),
        pltpu.VMEM((KP,), jnp.int32),
        pltpu.VMEM((KP,), jnp.float32),
    ],
)
def _gather_kernel(s_hbm, idx_hbm, l_hbm, table_v, idx_v, out_v):
    wid = lax.axis_index("s") * NC + lax.axis_index("c")
    for r in range(B_PER_W):
        b = wid * B_PER_W + r
        pltpu.sync_copy(idx_hbm.at[b], idx_v)
        pltpu.sync_copy(s_hbm.at[b], table_v)

        def body(c, _):
            iv = idx_v[pl.ds(c * 16, 16)]
            out_v[pl.ds(c * 16, 16)] = plsc.load_gather(table_v, [iv]) / T
            return 0

        lax.fori_loop(0, CHUNKS, body, 0, unroll=8)
        pltpu.sync_copy(out_v, l_hbm.at[b])


def _update_body(y_ref, x1_ref, x2_ref, m1_ref, m2_ref, r1_ref, r2_ref,
                 o1_ref, o2_ref):
    del y_ref, r1_ref, r2_ref
    for x_ref, m_ref, o_ref in ((x1_ref, m1_ref, o1_ref),
                                (x2_ref, m2_ref, o2_ref)):
        w = m_ref[...] * M + x_ref[...] * (1.0 - M)
        n = jnp.sqrt(jnp.sum(w * w, axis=2, keepdims=True))
        o_ref[...] = w / jnp.clip(n, 1e-12, None)


_update_call = pl.pallas_call(
    _update_body,
    grid_spec=pltpu.PrefetchScalarGridSpec(
        num_scalar_prefetch=1,
        grid=(BSZ,),
        in_specs=[
            pl.BlockSpec((1, 1, N_DIM), lambda i, y: (i, 0, 0)),
            pl.BlockSpec((1, 1, N_DIM), lambda i, y: (i, 0, 0)),
            pl.BlockSpec((1, 1, N_DIM), lambda i, y: (y[i], 0, 0)),
            pl.BlockSpec((1, 1, N_DIM), lambda i, y: (y[i], 0, 0)),
            pl.BlockSpec((1, 1, N_DIM), lambda i, y: (y[i], 0, 0)),
            pl.BlockSpec((1, 1, N_DIM), lambda i, y: (y[i], 0, 0)),
        ],
        out_specs=[
            pl.BlockSpec((1, 1, N_DIM), lambda i, y: (y[i], 0, 0)),
            pl.BlockSpec((1, 1, N_DIM), lambda i, y: (y[i], 0, 0)),
        ],
    ),
    out_shape=(
        jax.ShapeDtypeStruct((N_DATA, 1, N_DIM), jnp.float32),
        jax.ShapeDtypeStruct((N_DATA, 1, N_DIM), jnp.float32),
    ),
    input_output_aliases={5: 0, 6: 1},
)


def kernel(x1, x2, y, memory_1, memory_2, idx):
    idx_pad = jnp.pad(idx.at[:, 0].set(y), ((0, 0), (0, KP - (K + 1))))
    scores1, raw2 = _scores_call(x1, memory_2)
    l1p = _gather_kernel(scores1, idx_pad)
    scores2, raw1 = _scores_call(x2, memory_1)
    l2p = _gather_kernel(scores2, idx_pad)
    new1, new2 = _update_call(
        y,
        x1.reshape(BSZ, 1, N_DIM),
        x2.reshape(BSZ, 1, N_DIM),
        memory_1.reshape(N_DATA, 1, N_DIM),
        memory_2.reshape(N_DATA, 1, N_DIM),
        raw1, raw2)
    labels = jnp.zeros((BSZ,), jnp.int32)
    return (l1p[:, :K + 1], l2p[:, :K + 1], labels,
            new1.reshape(N_DATA, N_DIM), new2.reshape(N_DATA, N_DIM))


# speedup vs baseline: 1.0130x; 1.0085x over previous
import functools

import jax
import jax.numpy as jnp
from jax import lax
from jax.experimental import pallas as pl
from jax.experimental.pallas import tpu as pltpu
from jax.experimental.pallas import tpu_sc as plsc

BSZ = 64
N_DIM = 128
N_DATA = 100000
K = 8192
T = 0.07
M = 0.5

BLK = 8192
NBLK = (N_DATA + BLK - 1) // BLK
KP = 8208
CHUNKS = KP // 16
NC = 2
NS = 16
B_PER_W = BSZ // (NC * NS)


def _scores_body(x_ref, m_ref, s_ref, r_ref):
    dn = (((1,), (1,)), ((), ()))
    s_ref[...] = lax.dot_general(x_ref[...], m_ref[...], dn,
                                 preferred_element_type=jnp.float32)
    r_ref[...] = m_ref[...][:, None, :]


_scores_call = pl.pallas_call(
    _scores_body,
    grid=(NBLK,),
    in_specs=[
        pl.BlockSpec((BSZ, N_DIM), lambda i: (0, 0)),
        pl.BlockSpec((BLK, N_DIM), lambda i: (i, 0)),
    ],
    out_specs=[
        pl.BlockSpec((BSZ, BLK), lambda i: (0, i)),
        pl.BlockSpec((BLK, 1, N_DIM), lambda i: (i, 0, 0)),
    ],
    out_shape=(
        jax.ShapeDtypeStruct((BSZ, N_DATA), jnp.float32),
        jax.ShapeDtypeStruct((N_DATA, 1, N_DIM), jnp.float32),
    ),
)


_sc_mesh = plsc.VectorSubcoreMesh(
    core_axis_name="c", subcore_axis_name="s", num_cores=NC, num_subcores=NS)


@functools.partial(
    pl.kernel,
    out_type=jax.ShapeDtypeStruct((BSZ, KP), jnp.float32),
    mesh=_sc_mesh,
    compiler_params=pltpu.CompilerParams(needs_layout_passes=False),
    scratch_types=[
        pltpu.VMEM((N_DATA,), jnp.float32),
        pltpu.VMEM((KP,), jnp.int32),
        pltpu.VMEM((KP,), jnp.float32),
    ],
)
def _gather_kernel(s_hbm, idx_hbm, l_hbm, table_v, idx_v, out_v):
    wid = lax.axis_index("s") * NC + lax.axis_index("c")
    for r in range(B_PER_W):
        b = wid * B_PER_W + r
        pltpu.sync_copy(idx_hbm.at[b], idx_v)
        pltpu.sync_copy(s_hbm.at[b], table_v)

        def body(c, _):
            iv = idx_v[pl.ds(c * 16, 16)]
            out_v[pl.ds(c * 16, 16)] = plsc.load_gather(table_v, [iv]) / T
            return 0

        lax.fori_loop(0, CHUNKS, body, 0, unroll=8)
        pltpu.sync_copy(out_v, l_hbm.at[b])


def _update_body(y_ref, x1_ref, x2_ref, m1_ref, m2_ref, r1_ref, r2_ref,
                 o1_ref, o2_ref):
    del y_ref, r1_ref, r2_ref
    i = pl.program_id(0)
    for x_ref, m_ref, o_ref in ((x1_ref, m1_ref, o1_ref),
                                (x2_ref, m2_ref, o2_ref)):
        x = x_ref[pl.ds(i, 1)]
        w = m_ref[...] * M + x * (1.0 - M)
        n = jnp.sqrt(jnp.sum(w * w, axis=2, keepdims=True))
        o_ref[...] = w / jnp.clip(n, 1e-12, None)


_update_call = pl.pallas_call(
    _update_body,
    grid_spec=pltpu.PrefetchScalarGridSpec(
        num_scalar_prefetch=1,
        grid=(BSZ,),
        in_specs=[
            pl.BlockSpec((BSZ, 1, N_DIM), lambda i, y: (0, 0, 0)),
            pl.BlockSpec((BSZ, 1, N_DIM), lambda i, y: (0, 0, 0)),
            pl.BlockSpec((1, 1, N_DIM), lambda i, y: (y[i], 0, 0)),
            pl.BlockSpec((1, 1, N_DIM), lambda i, y: (y[i], 0, 0)),
            pl.BlockSpec(memory_space=pl.ANY),
            pl.BlockSpec(memory_space=pl.ANY),
        ],
        out_specs=[
            pl.BlockSpec((1, 1, N_DIM), lambda i, y: (y[i], 0, 0)),
            pl.BlockSpec((1, 1, N_DIM), lambda i, y: (y[i], 0, 0)),
        ],
    ),
    out_shape=(
        jax.ShapeDtypeStruct((N_DATA, 1, N_DIM), jnp.float32),
        jax.ShapeDtypeStruct((N_DATA, 1, N_DIM), jnp.float32),
    ),
    input_output_aliases={5: 0, 6: 1},
)


def kernel(x1, x2, y, memory_1, memory_2, idx):
    idx_pad = jnp.pad(idx.at[:, 0].set(y), ((0, 0), (0, KP - (K + 1))))
    scores1, raw2 = _scores_call(x1, memory_2)
    l1p = _gather_kernel(scores1, idx_pad)
    scores2, raw1 = _scores_call(x2, memory_1)
    l2p = _gather_kernel(scores2, idx_pad)
    new1, new2 = _update_call(
        y,
        x1.reshape(BSZ, 1, N_DIM),
        x2.reshape(BSZ, 1, N_DIM),
        memory_1.reshape(N_DATA, 1, N_DIM),
        memory_2.reshape(N_DATA, 1, N_DIM),
        raw1, raw2)
    labels = jnp.zeros((BSZ,), jnp.int32)
    return (l1p[:, :K + 1], l2p[:, :K + 1], labels,
            new1.reshape(N_DATA, N_DIM), new2.reshape(N_DATA, N_DIM))
